# baseline (speedup 1.0000x reference)
import functools

import jax
import jax.numpy as jnp
from jax.experimental import pallas as pl

_N_ITERS = 6
_ROW_BLOCK = 256


def _loss_kernel(x_ref, tgt_ref, out_ref, *, C: int):
    R = x_ref.shape[0]
    x = x_ref[...]
    col = jax.lax.broadcasted_iota(jnp.int32, (R, x.shape[1]), 1)
    valid = col < C
    x = jnp.where(valid, x, 0.0)
    neg_big = jnp.float32(-1e30)
    mx = jnp.max(jnp.where(valid, x, neg_big), axis=1, keepdims=True)
    Xs = jnp.where(valid, (x - mx) * 0.5, neg_big)

    tau = jnp.full((R, 1), -1.0, dtype=jnp.float32)
    for _ in range(_N_ITERS):
        r = jnp.maximum(Xs - tau, 0.0)
        s1 = jnp.sum(r, axis=1, keepdims=True)
        s2 = jnp.sum(r * r, axis=1, keepdims=True)
        k = jnp.sum((Xs > tau).astype(jnp.float32), axis=1, keepdims=True)
        ss = s2 - s1 * s1 / k
        delta = jnp.maximum((1.0 - ss) / k, 0.0)
        tau = tau + s1 / k - jnp.sqrt(delta)

    r = jnp.maximum(Xs - tau, 0.0)
    tgt = tgt_ref[0, 0, :].reshape(R, 1)
    tgt_val = jnp.sum(jnp.where(col == tgt, x, 0.0), axis=1)
    row_loss = ((1.0 - jnp.sum(r * r * r, axis=1)) / 0.75
                + jnp.sum(r * r * x, axis=1) - tgt_val)
    block_sum = jnp.sum(row_loss).reshape(1, 1)

    @pl.when(pl.program_id(0) == 0)
    def _():
        out_ref[...] = jnp.zeros((1, 1), jnp.float32)

    out_ref[...] += block_sum


@jax.jit
def kernel(input, target):
    n, C = input.shape
    R = _ROW_BLOCK
    nb = n // R
    tgt3 = target.astype(jnp.int32).reshape(nb, 1, R)
    total = pl.pallas_call(
        functools.partial(_loss_kernel, C=C),
        grid=(nb,),
        in_specs=[
            pl.BlockSpec((R, C), lambda i: (i, 0)),
            pl.BlockSpec((1, 1, R), lambda i: (i, 0, 0)),
        ],
        out_specs=pl.BlockSpec((1, 1), lambda i: (0, 0)),
        out_shape=jax.ShapeDtypeStruct((1, 1), jnp.float32),
    )(input, tgt3)
    return total[0, 0] / jnp.float32(n)


# speedup vs baseline: 1.1458x; 1.1458x over previous
import functools

import jax
import jax.numpy as jnp
from jax.experimental import pallas as pl

_N_ITERS = 5
_ROW_BLOCK = 512


def _loss_kernel(x_ref, tgt_ref, out_ref, *, C: int):
    R = x_ref.shape[0]
    x = x_ref[...]
    col = jax.lax.broadcasted_iota(jnp.int32, (R, x.shape[1]), 1)
    valid = col < C
    x = jnp.where(valid, x, 0.0)
    neg_big = jnp.float32(-1e30)
    mx = jnp.max(jnp.where(valid, x, neg_big), axis=1, keepdims=True)
    Xs = jnp.where(valid, (x - mx) * 0.5, neg_big)

    tau = jnp.full((R, 1), -1.0, dtype=jnp.float32)
    for _ in range(_N_ITERS):
        r = jnp.maximum(Xs - tau, 0.0)
        s1 = jnp.sum(r, axis=1, keepdims=True)
        s2 = jnp.sum(r * r, axis=1, keepdims=True)
        k = jnp.sum((Xs > tau).astype(jnp.float32), axis=1, keepdims=True)
        ss = s2 - s1 * s1 / k
        delta = jnp.maximum((1.0 - ss) / k, 0.0)
        tau = tau + s1 / k - jnp.sqrt(delta)

    r = jnp.maximum(Xs - tau, 0.0)
    tgt = tgt_ref[0, 0, :].reshape(R, 1)
    tgt_val = jnp.sum(jnp.where(col == tgt, x, 0.0), axis=1)
    row_loss = ((1.0 - jnp.sum(r * r * r, axis=1)) / 0.75
                + jnp.sum(r * r * x, axis=1) - tgt_val)
    block_sum = jnp.sum(row_loss).reshape(1, 1)

    @pl.when(pl.program_id(0) == 0)
    def _():
        out_ref[...] = jnp.zeros((1, 1), jnp.float32)

    out_ref[...] += block_sum


@jax.jit
def kernel(input, target):
    n, C = input.shape
    R = _ROW_BLOCK
    nb = n // R
    tgt3 = target.astype(jnp.int32).reshape(nb, 1, R)
    total = pl.pallas_call(
        functools.partial(_loss_kernel, C=C),
        grid=(nb,),
        in_specs=[
            pl.BlockSpec((R, C), lambda i: (i, 0)),
            pl.BlockSpec((1, 1, R), lambda i: (i, 0, 0)),
        ],
        out_specs=pl.BlockSpec((1, 1), lambda i: (0, 0)),
        out_shape=jax.ShapeDtypeStruct((1, 1), jnp.float32),
    )(input, tgt3)
    return total[0, 0] / jnp.float32(n)


# speedup vs baseline: 1.1548x; 1.0078x over previous
import functools

import jax
import jax.numpy as jnp
from jax.experimental import pallas as pl

_N_ITERS = 5
_ROW_BLOCK = 1024


def _loss_kernel(x_ref, tgt_ref, out_ref, *, C: int):
    R = x_ref.shape[0]
    x = x_ref[...]
    col = jax.lax.broadcasted_iota(jnp.int32, (R, x.shape[1]), 1)
    valid = col < C
    x = jnp.where(valid, x, 0.0)
    neg_big = jnp.float32(-1e30)
    mx = jnp.max(jnp.where(valid, x, neg_big), axis=1, keepdims=True)
    Xs = jnp.where(valid, (x - mx) * 0.5, neg_big)

    tau = jnp.full((R, 1), -1.0, dtype=jnp.float32)
    for _ in range(_N_ITERS):
        r = jnp.maximum(Xs - tau, 0.0)
        s1 = jnp.sum(r, axis=1, keepdims=True)
        s2 = jnp.sum(r * r, axis=1, keepdims=True)
        k = jnp.sum((Xs > tau).astype(jnp.float32), axis=1, keepdims=True)
        ss = s2 - s1 * s1 / k
        delta = jnp.maximum((1.0 - ss) / k, 0.0)
        tau = tau + s1 / k - jnp.sqrt(delta)

    r = jnp.maximum(Xs - tau, 0.0)
    tgt = tgt_ref[0, 0, :].reshape(R, 1)
    tgt_val = jnp.sum(jnp.where(col == tgt, x, 0.0), axis=1)
    row_loss = ((1.0 - jnp.sum(r * r * r, axis=1)) / 0.75
                + jnp.sum(r * r * x, axis=1) - tgt_val)
    block_sum = jnp.sum(row_loss).reshape(1, 1)

    @pl.when(pl.program_id(0) == 0)
    def _():
        out_ref[...] = jnp.zeros((1, 1), jnp.float32)

    out_ref[...] += block_sum


@jax.jit
def kernel(input, target):
    n, C = input.shape
    R = _ROW_BLOCK
    nb = n // R
    tgt3 = target.astype(jnp.int32).reshape(nb, 1, R)
    total = pl.pallas_call(
        functools.partial(_loss_kernel, C=C),
        grid=(nb,),
        in_specs=[
            pl.BlockSpec((R, C), lambda i: (i, 0)),
            pl.BlockSpec((1, 1, R), lambda i: (i, 0, 0)),
        ],
        out_specs=pl.BlockSpec((1, 1), lambda i: (0, 0)),
        out_shape=jax.ShapeDtypeStruct((1, 1), jnp.float32),
    )(input, tgt3)
    return total[0, 0] / jnp.float32(n)
